# baseline (speedup 1.0000x reference)
import functools

import jax
import jax.numpy as jnp
from jax import lax
from jax.experimental import pallas as pl
from jax.experimental.pallas import tpu as pltpu
from jax.experimental.pallas import tpu_sc as plsc

N = 10000
E = 320000
D = 128
R = 1000
NBLK = N // R

_NC, _NS, _NL = 2, 16, 16
_NW = _NC * _NS

_f32 = jnp.float32


def _make_spmm():
    CH = 128
    NCHT = E // CH
    NMAIN = NCHT // _NW
    NEXTRA = NCHT - NMAIN * _NW
    CR = 200
    NCR = N // CR
    ZR = 40
    mesh = plsc.VectorSubcoreMesh(core_axis_name="c", subcore_axis_name="s")

    @functools.partial(
        pl.kernel,
        out_type=jax.ShapeDtypeStruct((2, N, D), _f32),
        mesh=mesh,
        compiler_params=pltpu.CompilerParams(needs_layout_passes=False),
        scratch_types=[
            pltpu.VMEM((3, CH), jnp.int32),
            pltpu.VMEM((3, CH), jnp.int32),
            pltpu.VMEM((CH, D), _f32),
            pltpu.VMEM((CH, D), _f32),
            pltpu.VMEM((ZR, D), _f32),
            pltpu.VMEM_SHARED((N, D), _f32),
            pltpu.SemaphoreType.DMA,
            pltpu.SemaphoreType.DMA,
        ],
    )
    def spmm(table, epk, out, pkd0, pkd1, rows0, rows1, zb, acc,
             sem, sem_sc):
        pkds = (pkd0, pkd1)
        rowss = (rows0, rows1)
        cid = lax.axis_index("c")
        sid = lax.axis_index("s")
        wid = sid * _NC + cid
        def start(slot, c, wait_scat=True):
            if wait_scat:
                pltpu.make_async_copy(
                    rowss[slot], acc.at[pkds[slot].at[1]], sem_sc).wait()
            cb = pl.multiple_of(c * CH, 8)
            pltpu.sync_copy(epk.at[:, pl.ds(cb, CH)], pkds[slot])
            for h in range(4):
                q = CH // 4
                pltpu.async_copy(
                    table.at[pkds[slot].at[0, pl.ds(h * q, q)]],
                    rowss[slot].at[pl.ds(h * q, q)], sem)

        def process(slot):
            pkd = pkds[slot]
            rows = rowss[slot]
            for h in range(4):
                q = CH // 4
                pltpu.make_async_copy(
                    table.at[pkd.at[0, pl.ds(h * q, q)]],
                    rows.at[pl.ds(h * q, q)], sem).wait()

            def scale(t, c2):
                ridx = lax.iota(jnp.int32, _NL) + t * _NL
                wv = plsc.bitcast(pkd[2, pl.ds(t * _NL, _NL)], _f32)
                for col in range(D):
                    cidx = jnp.full((_NL,), col, jnp.int32)
                    v = plsc.load_gather(rows, [ridx, cidx])
                    plsc.store_scatter(rows, [ridx, cidx], v * wv)
                return c2

            lax.fori_loop(0, CH // _NL, scale, 0)
            pltpu.async_copy(rows, acc.at[pkd.at[1]], sem_sc, add=True)

        def drain(slot):
            pltpu.make_async_copy(
                rowss[slot], acc.at[pkds[slot].at[1]], sem_sc).wait()

        c0 = wid * NMAIN
        start(0, c0, wait_scat=False)
        start(1, c0 + 1, wait_scat=False)
        zv = jnp.zeros((_NL,), _f32)
        for i in range(ZR):
            for k in range(D // _NL):
                zb[i, pl.ds(k * _NL, _NL)] = zv
        for j in range((NCR + _NS - 1) // _NS):
            ci = sid + j * _NS

            @pl.when(ci < NCR)
            def _():
                rb = pl.multiple_of(ci * CR, 8)
                for k in range(CR // ZR):
                    pltpu.sync_copy(zb, acc.at[pl.ds(rb + k * ZR, ZR)])
        plsc.subcore_barrier()

        def pair(t, carry):
            for s in range(2):
                process(s)
                start(s, c0 + 2 * t + 2 + s)
            return carry

        lax.fori_loop(0, NMAIN // 2 - 1, pair, 0)
        for s in range(2):
            process(s)
        for s in range(2):
            drain(s)

        @pl.when(wid < NEXTRA)
        def _():
            start(0, NMAIN * _NW + wid, wait_scat=False)
            process(0)
            drain(0)

        plsc.subcore_barrier()
        for j in range((NCR + _NS - 1) // _NS):
            ci = sid + j * _NS

            @pl.when(ci < NCR)
            def _():
                rb = pl.multiple_of(ci * CR, 8)
                pltpu.sync_copy(acc.at[pl.ds(rb, CR)],
                                out.at[cid, pl.ds(rb, CR)])

    return spmm


_spmm = _make_spmm()


def _make_gather(B):
    per = B // _NW
    mesh = plsc.VectorSubcoreMesh(core_axis_name="c", subcore_axis_name="s")

    @functools.partial(
        pl.kernel,
        out_type=jax.ShapeDtypeStruct((B, D), _f32),
        mesh=mesh,
        compiler_params=pltpu.CompilerParams(needs_layout_passes=False),
        scratch_types=[
            pltpu.VMEM((per,), jnp.int32),
            pltpu.VMEM((per, D), _f32),
            pltpu.SemaphoreType.DMA,
        ],
    )
    def gath(table, idx, out, idx_v, rows_v, sem):
        cid = lax.axis_index("c")
        sid = lax.axis_index("s")
        base = (sid * _NC + cid) * per
        pltpu.sync_copy(idx.at[pl.ds(base, per)], idx_v)
        pltpu.async_copy(table.at[idx_v], rows_v, sem).wait()
        pltpu.sync_copy(rows_v, out.at[pl.ds(base, per)])

    return gath


_gather1024 = _make_gather(1024)


def _mm2_body(x1, x2, w, o1, o2):
    o1[...] = jnp.dot(x1[...], w[...], preferred_element_type=_f32)
    o2[...] = jnp.dot(x2[...], w[...], preferred_element_type=_f32)


def _row_spec():
    return pl.BlockSpec((R, D), lambda i: (i, 0))


def _pair_spec():
    return pl.BlockSpec((2, R, D), lambda i: (0, i, 0))


def _full_spec(s):
    return pl.BlockSpec(s, lambda i: tuple(0 for _ in s))


def _kb_body(u1, ua, ub, u2, b0r, a0r, w1, v1, va, vb, v2, stats):
    i = pl.program_id(0)

    def act(u):
        h = u[0] + u[1] + b0r[...]
        return jnp.where(h > 0, h, a0r[...] * h)

    x1 = act(u1)
    xa = act(ua)
    xb = act(ub)
    x2 = act(u2)
    v1[...] = jnp.dot(x1, w1[...], preferred_element_type=_f32)
    va[...] = jnp.dot(xa, w1[...], preferred_element_type=_f32)
    vb[...] = jnp.dot(xb, w1[...], preferred_element_type=_f32)
    v2[...] = jnp.dot(x2, w1[...], preferred_element_type=_f32)
    s = jnp.sum(x1, axis=0, keepdims=True)
    sq = jnp.sum(x1 * x1, axis=0, keepdims=True)
    st = jnp.concatenate([s, sq], axis=0)

    @pl.when(i == 0)
    def _():
        stats[...] = st

    @pl.when(i > 0)
    def _():
        stats[...] = stats[...] + st


def _blp_body(u1, stats, b0r, a0r, g0r, be0r, w1, vlp):
    h = u1[0] + u1[1] + b0r[...]
    x = jnp.where(h > 0, h, a0r[...] * h)
    mu = stats[0:1, :] * (1.0 / N)
    var = stats[1:2, :] * (1.0 / N) - mu * mu
    sc = g0r[...] * lax.rsqrt(var + 1e-5)
    xl = (x - mu) * sc + be0r[...]
    vlp[...] = jnp.dot(xl, w1[...], preferred_element_type=_f32)


def _cs_body(z1, za, zb, zl, b1r, a1r, stats):
    i = pl.program_id(0)

    def act(z):
        h = z[0] + z[1] + b1r[...]
        return jnp.where(h > 0, h, a1r[...] * h)

    h1 = act(z1)
    ha = act(za)
    hb = act(zb)
    hl = act(zl)
    rows = [
        jnp.sum(h1, axis=0, keepdims=True),
        jnp.sum(ha, axis=0, keepdims=True),
        jnp.sum(hb, axis=0, keepdims=True),
        jnp.sum(hl, axis=0, keepdims=True),
        jnp.sum(hl * hl, axis=0, keepdims=True),
    ]
    st = jnp.concatenate(rows + [jnp.zeros((3, D), _f32)], axis=0)

    @pl.when(i == 0)
    def _():
        stats[...] = st

    @pl.when(i > 0)
    def _():
        stats[...] = stats[...] + st


def _bce(x, y):
    return jnp.maximum(x, 0.0) - x * y + jnp.log1p(jnp.exp(-jnp.abs(x)))


def _cm_body(z1, z2, zl, stats, b1r, a1r, g1r, be1r, dgip, dgiW, clp, clW,
             lpp, lblT, scal, out3, lacc):
    i = pl.program_id(0)

    def act(z):
        h = z[0] + z[1] + b1r[...]
        return jnp.where(h > 0, h, a1r[...] * h)

    h1 = act(z1)
    h2 = act(z2)
    hlp = act(zl)
    c = jax.nn.sigmoid(stats[0:1, :] * (1.0 / N))
    c1 = jax.nn.sigmoid(stats[1:2, :] * (1.0 / N) * clp[...])
    c3 = jax.nn.sigmoid(stats[2:3, :] * (1.0 / N) * clp[...])
    db = scal[0]
    cb = scal[1]
    y1 = lblT[:, 0:1]
    y2 = lblT[:, 1:2]
    d1 = jnp.dot(h1 * dgip[...], dgiW[...], preferred_element_type=_f32)
    d2 = jnp.dot(h2 * dgip[...], dgiW[...], preferred_element_type=_f32)
    sc1 = jnp.sum(d1 * c, axis=1, keepdims=True) + db
    sc2 = jnp.sum(d2 * c, axis=1, keepdims=True) + db
    dsum = jnp.sum(_bce(sc1, y1)) + jnp.sum(_bce(sc2, y2))
    cc = c1 + c3
    e1 = jnp.dot(h1 * clp[...], clW[...], preferred_element_type=_f32)
    e2 = jnp.dot(h2 * clp[...], clW[...], preferred_element_type=_f32)
    st1 = jnp.sum(e1 * cc, axis=1, keepdims=True) + 2.0 * cb
    st2 = jnp.sum(e2 * cc, axis=1, keepdims=True) + 2.0 * cb
    csum = jnp.sum(_bce(st1, y1)) + jnp.sum(_bce(st2, y2))
    mu2 = stats[3:4, :] * (1.0 / N)
    var2 = stats[4:5, :] * (1.0 / N) - mu2 * mu2
    hl = (hlp - mu2) * (g1r[...] * lax.rsqrt(var2 + 1e-5)) + be1r[...]
    fv = hl * lpp[...]
    out3[...] = jnp.where(fv > 0, fv, jnp.exp(fv) - 1.0)

    @pl.when(i == 0)
    def _():
        lacc[0] = dsum
        lacc[1] = csum

    @pl.when(i > 0)
    def _():
        lacc[0] = lacc[0] + dsum
        lacc[1] = lacc[1] + csum


def _kd_body(f100, g3, lacc, out):
    fi = f100[...]
    g = g3[...]
    num = jnp.sum(fi[:, None, :] * g, axis=2)
    na = jnp.sqrt(jnp.sum(fi * fi, axis=1, keepdims=True))
    nb = jnp.sqrt(jnp.sum(g * g, axis=2))
    eps = 1e-8
    sim = num / (jnp.maximum(na, eps) * jnp.maximum(nb, eps))
    ex = jnp.exp(sim) / 1.5
    lane = lax.broadcasted_iota(jnp.int32, ex.shape, 1)
    numv = jnp.sum(jnp.where(lane == 0, ex, 0.0), axis=1, keepdims=True)
    denv = jnp.sum(jnp.where(lane >= 1, ex, 0.0), axis=1, keepdims=True)
    lp = jnp.mean(-jnp.log(numv / denv))
    dgi = lacc[0] / (2.0 * N)
    cl = lacc[1] / (2.0 * N)
    out[0, 0] = 0.5 * dgi + 0.3 * cl + 0.2 * lp


def kernel(seq1, seq2, seq3, seq4, edge_w, aug1_w, aug2_w, lbl, W0, b0,
           prelu0, W1, b1, prelu1, gamma0, beta0, gamma1, beta1, dgi_prompt,
           dgi_W, dgi_b, cl_prompt, cl_W, cl_b, lp_prompt, edge_index,
           aug1_edge_index, aug2_edge_index, sample):
    x1 = seq1[0]
    x2 = seq2[0]

    def pack(ei, w):
        wb = lax.bitcast_convert_type(w, jnp.int32)
        return jnp.concatenate([ei, wb[None, :]], axis=0)

    epk_e = pack(edge_index, edge_w)
    epk_a1 = pack(aug1_edge_index, aug1_w)
    epk_a2 = pack(aug2_edge_index, aug2_w)

    b0r = b0.reshape(1, D)
    b1r = b1.reshape(1, D)
    a0r = jnp.broadcast_to(prelu0.reshape(1, 1), (1, D))
    a1r = jnp.broadcast_to(prelu1.reshape(1, 1), (1, D))
    g0r = gamma0.reshape(1, D)
    be0r = beta0.reshape(1, D)
    g1r = gamma1.reshape(1, D)
    be1r = beta1.reshape(1, D)
    lblT = lbl.reshape(2, N).T
    scal = jnp.stack([dgi_b[0], cl_b[0]])

    t1, t2 = pl.pallas_call(
        _mm2_body,
        grid=(NBLK,),
        in_specs=[_row_spec(), _row_spec(), _full_spec((D, D))],
        out_specs=[_row_spec(), _row_spec()],
        out_shape=[jax.ShapeDtypeStruct((N, D), _f32)] * 2,
    )(x1, x2, W0)

    p_s1e = _spmm(t1, epk_e)
    p_a1 = _spmm(t1, epk_a1)
    p_a2 = _spmm(t1, epk_a2)
    p_s2e = _spmm(t2, epk_e)

    v1, va, vb, v2, stats0 = pl.pallas_call(
        _kb_body,
        grid=(NBLK,),
        in_specs=[_pair_spec()] * 4 + [_full_spec((1, D)), _full_spec((1, D)),
                                       _full_spec((D, D))],
        out_specs=[_row_spec()] * 4 + [_full_spec((2, D))],
        out_shape=[jax.ShapeDtypeStruct((N, D), _f32)] * 4
        + [jax.ShapeDtypeStruct((2, D), _f32)],
    )(p_s1e, p_a1, p_a2, p_s2e, b0r, a0r, W1)

    vlp = pl.pallas_call(
        _blp_body,
        grid=(NBLK,),
        in_specs=[_pair_spec(), _full_spec((2, D))] + [_full_spec((1, D))] * 4
        + [_full_spec((D, D))],
        out_specs=_row_spec(),
        out_shape=jax.ShapeDtypeStruct((N, D), _f32),
    )(p_s1e, stats0, b0r, a0r, g0r, be0r, W1)

    q_s1e = _spmm(v1, epk_e)
    q_a1 = _spmm(va, epk_a1)
    q_a2 = _spmm(vb, epk_a2)
    q_s2e = _spmm(v2, epk_e)
    q_lp = _spmm(vlp, epk_e)

    stats1 = pl.pallas_call(
        _cs_body,
        grid=(NBLK,),
        in_specs=[_pair_spec()] * 4 + [_full_spec((1, D))] * 2,
        out_specs=_full_spec((8, D)),
        out_shape=jax.ShapeDtypeStruct((8, D), _f32),
    )(q_s1e, q_a1, q_a2, q_lp, b1r, a1r)

    logits3, lacc = pl.pallas_call(
        _cm_body,
        grid=(NBLK,),
        in_specs=[_pair_spec()] * 3 + [_full_spec((8, D))]
        + [_full_spec((1, D))] * 4
        + [_full_spec((1, D)), _full_spec((D, D)), _full_spec((1, D)),
           _full_spec((D, D)), _full_spec((1, D))]
        + [pl.BlockSpec((R, 2), lambda i: (i, 0)),
           pl.BlockSpec(memory_space=pltpu.SMEM)],
        out_specs=[_row_spec(), pl.BlockSpec(memory_space=pltpu.SMEM)],
        out_shape=[jax.ShapeDtypeStruct((N, D), _f32),
                   jax.ShapeDtypeStruct((2,), _f32)],
    )(q_s1e, q_s2e, q_lp, stats1, b1r, a1r, g1r, be1r,
      dgi_prompt, dgi_W, cl_prompt, cl_W, lp_prompt, lblT, scal)

    samp = jnp.concatenate(
        [sample.reshape(-1), jnp.zeros((24,), jnp.int32)])
    g1024 = _gather1024(logits3, samp)
    g3 = g1024[:1000].reshape(100, 10, D)
    f100 = logits3[:100]

    out = pl.pallas_call(
        _kd_body,
        in_specs=[pl.BlockSpec(memory_space=pltpu.VMEM),
                  pl.BlockSpec(memory_space=pltpu.VMEM),
                  pl.BlockSpec(memory_space=pltpu.SMEM)],
        out_specs=pl.BlockSpec(memory_space=pltpu.SMEM),
        out_shape=jax.ShapeDtypeStruct((1, 1), _f32),
    )(f100, g3, lacc)
    return out.reshape(())


# speedup vs baseline: 7.7236x; 7.7236x over previous
import functools

import jax
import jax.numpy as jnp
from jax import lax
from jax.experimental import pallas as pl
from jax.experimental.pallas import tpu as pltpu
from jax.experimental.pallas import tpu_sc as plsc

N = 10000
E = 320000
D = 128
R = 1000
NBLK = N // R

_NC, _NS, _NL = 2, 16, 16
_NW = _NC * _NS

_f32 = jnp.float32


def _make_spmm():
    CH = 128
    NCHT = E // CH
    NMAIN = NCHT // _NW
    NEXTRA = NCHT - NMAIN * _NW
    CR = 200
    NCR = N // CR
    ZR = 40
    mesh = plsc.VectorSubcoreMesh(core_axis_name="c", subcore_axis_name="s")

    @functools.partial(
        pl.kernel,
        out_type=jax.ShapeDtypeStruct((2, N, D), _f32),
        mesh=mesh,
        compiler_params=pltpu.CompilerParams(needs_layout_passes=False),
        scratch_types=[
            pltpu.VMEM((3, CH), jnp.int32),
            pltpu.VMEM((3, CH), jnp.int32),
            pltpu.VMEM((CH, D), _f32),
            pltpu.VMEM((CH, D), _f32),
            pltpu.VMEM((ZR, D), _f32),
            pltpu.VMEM_SHARED((N, D), _f32),
            pltpu.SemaphoreType.DMA,
            pltpu.SemaphoreType.DMA,
        ],
    )
    def spmm(table, epk, out, pkd0, pkd1, rows0, rows1, zb, acc,
             sem, sem_sc):
        pkds = (pkd0, pkd1)
        rowss = (rows0, rows1)
        cid = lax.axis_index("c")
        sid = lax.axis_index("s")
        wid = sid * _NC + cid
        def start(slot, c, wait_scat=True):
            if wait_scat:
                pltpu.make_async_copy(
                    rowss[slot], acc.at[pkds[slot].at[1]], sem_sc).wait()
            cb = pl.multiple_of(c * CH, 8)
            pltpu.sync_copy(epk.at[:, pl.ds(cb, CH)], pkds[slot])
            for h in range(4):
                q = CH // 4
                pltpu.async_copy(
                    table.at[pkds[slot].at[0, pl.ds(h * q, q)]],
                    rowss[slot].at[pl.ds(h * q, q)], sem)

        def process(slot):
            pkd = pkds[slot]
            rows = rowss[slot]
            for h in range(4):
                q = CH // 4
                pltpu.make_async_copy(
                    table.at[pkd.at[0, pl.ds(h * q, q)]],
                    rows.at[pl.ds(h * q, q)], sem).wait()

            def scale(t, c2):
                eb = pl.multiple_of(t * 32, 32)
                for u in range(32):
                    e = eb + u
                    ws = plsc.bitcast(
                        plsc.load_gather(
                            pkd,
                            [jnp.full((_NL,), 2, jnp.int32),
                             jnp.full((_NL,), u, jnp.int32) + eb]),
                        _f32)
                    for k in range(D // _NL):
                        rows[e, pl.ds(k * _NL, _NL)] = (
                            rows[e, pl.ds(k * _NL, _NL)] * ws)
                return c2

            lax.fori_loop(0, CH // 32, scale, 0)
            pltpu.async_copy(rows, acc.at[pkd.at[1]], sem_sc, add=True)

        def drain(slot):
            pltpu.make_async_copy(
                rowss[slot], acc.at[pkds[slot].at[1]], sem_sc).wait()

        c0 = wid * NMAIN
        start(0, c0, wait_scat=False)
        start(1, c0 + 1, wait_scat=False)
        zv = jnp.zeros((_NL,), _f32)
        for i in range(ZR):
            for k in range(D // _NL):
                zb[i, pl.ds(k * _NL, _NL)] = zv
        for j in range((NCR + _NS - 1) // _NS):
            ci = sid + j * _NS

            @pl.when(ci < NCR)
            def _():
                rb = pl.multiple_of(ci * CR, 8)
                for k in range(CR // ZR):
                    pltpu.sync_copy(zb, acc.at[pl.ds(rb + k * ZR, ZR)])
        plsc.subcore_barrier()

        def pair(t, carry):
            for s in range(2):
                process(s)
                start(s, c0 + 2 * t + 2 + s)
            return carry

        lax.fori_loop(0, NMAIN // 2 - 1, pair, 0)
        for s in range(2):
            process(s)
        for s in range(2):
            drain(s)

        @pl.when(wid < NEXTRA)
        def _():
            start(0, NMAIN * _NW + wid, wait_scat=False)
            process(0)
            drain(0)

        plsc.subcore_barrier()
        for j in range((NCR + _NS - 1) // _NS):
            ci = sid + j * _NS

            @pl.when(ci < NCR)
            def _():
                rb = pl.multiple_of(ci * CR, 8)
                pltpu.sync_copy(acc.at[pl.ds(rb, CR)],
                                out.at[cid, pl.ds(rb, CR)])

    return spmm


_spmm = _make_spmm()


def _make_gather(B):
    per = B // _NW
    mesh = plsc.VectorSubcoreMesh(core_axis_name="c", subcore_axis_name="s")

    @functools.partial(
        pl.kernel,
        out_type=jax.ShapeDtypeStruct((B, D), _f32),
        mesh=mesh,
        compiler_params=pltpu.CompilerParams(needs_layout_passes=False),
        scratch_types=[
            pltpu.VMEM((per,), jnp.int32),
            pltpu.VMEM((per, D), _f32),
            pltpu.SemaphoreType.DMA,
        ],
    )
    def gath(table, idx, out, idx_v, rows_v, sem):
        cid = lax.axis_index("c")
        sid = lax.axis_index("s")
        base = (sid * _NC + cid) * per
        pltpu.sync_copy(idx.at[pl.ds(base, per)], idx_v)
        pltpu.async_copy(table.at[idx_v], rows_v, sem).wait()
        pltpu.sync_copy(rows_v, out.at[pl.ds(base, per)])

    return gath


_gather1024 = _make_gather(1024)


def _mm2_body(x1, x2, w, o1, o2):
    o1[...] = jnp.dot(x1[...], w[...], preferred_element_type=_f32)
    o2[...] = jnp.dot(x2[...], w[...], preferred_element_type=_f32)


def _row_spec():
    return pl.BlockSpec((R, D), lambda i: (i, 0))


def _pair_spec():
    return pl.BlockSpec((2, R, D), lambda i: (0, i, 0))


def _full_spec(s):
    return pl.BlockSpec(s, lambda i: tuple(0 for _ in s))


def _kb_body(u1, ua, ub, u2, b0r, a0r, w1, v1, va, vb, v2, stats):
    i = pl.program_id(0)

    def act(u):
        h = u[0] + u[1] + b0r[...]
        return jnp.where(h > 0, h, a0r[...] * h)

    x1 = act(u1)
    xa = act(ua)
    xb = act(ub)
    x2 = act(u2)
    v1[...] = jnp.dot(x1, w1[...], preferred_element_type=_f32)
    va[...] = jnp.dot(xa, w1[...], preferred_element_type=_f32)
    vb[...] = jnp.dot(xb, w1[...], preferred_element_type=_f32)
    v2[...] = jnp.dot(x2, w1[...], preferred_element_type=_f32)
    s = jnp.sum(x1, axis=0, keepdims=True)
    sq = jnp.sum(x1 * x1, axis=0, keepdims=True)
    st = jnp.concatenate([s, sq], axis=0)

    @pl.when(i == 0)
    def _():
        stats[...] = st

    @pl.when(i > 0)
    def _():
        stats[...] = stats[...] + st


def _blp_body(u1, stats, b0r, a0r, g0r, be0r, w1, vlp):
    h = u1[0] + u1[1] + b0r[...]
    x = jnp.where(h > 0, h, a0r[...] * h)
    mu = stats[0:1, :] * (1.0 / N)
    var = stats[1:2, :] * (1.0 / N) - mu * mu
    sc = g0r[...] * lax.rsqrt(var + 1e-5)
    xl = (x - mu) * sc + be0r[...]
    vlp[...] = jnp.dot(xl, w1[...], preferred_element_type=_f32)


def _cs_body(z1, za, zb, zl, b1r, a1r, stats):
    i = pl.program_id(0)

    def act(z):
        h = z[0] + z[1] + b1r[...]
        return jnp.where(h > 0, h, a1r[...] * h)

    h1 = act(z1)
    ha = act(za)
    hb = act(zb)
    hl = act(zl)
    rows = [
        jnp.sum(h1, axis=0, keepdims=True),
        jnp.sum(ha, axis=0, keepdims=True),
        jnp.sum(hb, axis=0, keepdims=True),
        jnp.sum(hl, axis=0, keepdims=True),
        jnp.sum(hl * hl, axis=0, keepdims=True),
    ]
    st = jnp.concatenate(rows + [jnp.zeros((3, D), _f32)], axis=0)

    @pl.when(i == 0)
    def _():
        stats[...] = st

    @pl.when(i > 0)
    def _():
        stats[...] = stats[...] + st


def _bce(x, y):
    return jnp.maximum(x, 0.0) - x * y + jnp.log1p(jnp.exp(-jnp.abs(x)))


def _cm_body(z1, z2, zl, stats, b1r, a1r, g1r, be1r, dgip, dgiW, clp, clW,
             lpp, lblT, scal, out3, lacc):
    i = pl.program_id(0)

    def act(z):
        h = z[0] + z[1] + b1r[...]
        return jnp.where(h > 0, h, a1r[...] * h)

    h1 = act(z1)
    h2 = act(z2)
    hlp = act(zl)
    c = jax.nn.sigmoid(stats[0:1, :] * (1.0 / N))
    c1 = jax.nn.sigmoid(stats[1:2, :] * (1.0 / N) * clp[...])
    c3 = jax.nn.sigmoid(stats[2:3, :] * (1.0 / N) * clp[...])
    db = scal[0]
    cb = scal[1]
    y1 = lblT[:, 0:1]
    y2 = lblT[:, 1:2]
    d1 = jnp.dot(h1 * dgip[...], dgiW[...], preferred_element_type=_f32)
    d2 = jnp.dot(h2 * dgip[...], dgiW[...], preferred_element_type=_f32)
    sc1 = jnp.sum(d1 * c, axis=1, keepdims=True) + db
    sc2 = jnp.sum(d2 * c, axis=1, keepdims=True) + db
    dsum = jnp.sum(_bce(sc1, y1)) + jnp.sum(_bce(sc2, y2))
    cc = c1 + c3
    e1 = jnp.dot(h1 * clp[...], clW[...], preferred_element_type=_f32)
    e2 = jnp.dot(h2 * clp[...], clW[...], preferred_element_type=_f32)
    st1 = jnp.sum(e1 * cc, axis=1, keepdims=True) + 2.0 * cb
    st2 = jnp.sum(e2 * cc, axis=1, keepdims=True) + 2.0 * cb
    csum = jnp.sum(_bce(st1, y1)) + jnp.sum(_bce(st2, y2))
    mu2 = stats[3:4, :] * (1.0 / N)
    var2 = stats[4:5, :] * (1.0 / N) - mu2 * mu2
    hl = (hlp - mu2) * (g1r[...] * lax.rsqrt(var2 + 1e-5)) + be1r[...]
    fv = hl * lpp[...]
    out3[...] = jnp.where(fv > 0, fv, jnp.exp(fv) - 1.0)

    @pl.when(i == 0)
    def _():
        lacc[0] = dsum
        lacc[1] = csum

    @pl.when(i > 0)
    def _():
        lacc[0] = lacc[0] + dsum
        lacc[1] = lacc[1] + csum


def _kd_body(f100, g3, lacc, out):
    fi = f100[...]
    g = g3[...]
    num = jnp.sum(fi[:, None, :] * g, axis=2)
    na = jnp.sqrt(jnp.sum(fi * fi, axis=1, keepdims=True))
    nb = jnp.sqrt(jnp.sum(g * g, axis=2))
    eps = 1e-8
    sim = num / (jnp.maximum(na, eps) * jnp.maximum(nb, eps))
    ex = jnp.exp(sim) / 1.5
    lane = lax.broadcasted_iota(jnp.int32, ex.shape, 1)
    numv = jnp.sum(jnp.where(lane == 0, ex, 0.0), axis=1, keepdims=True)
    denv = jnp.sum(jnp.where(lane >= 1, ex, 0.0), axis=1, keepdims=True)
    lp = jnp.mean(-jnp.log(numv / denv))
    dgi = lacc[0] / (2.0 * N)
    cl = lacc[1] / (2.0 * N)
    out[0, 0] = 0.5 * dgi + 0.3 * cl + 0.2 * lp


def kernel(seq1, seq2, seq3, seq4, edge_w, aug1_w, aug2_w, lbl, W0, b0,
           prelu0, W1, b1, prelu1, gamma0, beta0, gamma1, beta1, dgi_prompt,
           dgi_W, dgi_b, cl_prompt, cl_W, cl_b, lp_prompt, edge_index,
           aug1_edge_index, aug2_edge_index, sample):
    x1 = seq1[0]
    x2 = seq2[0]

    def pack(ei, w):
        wb = lax.bitcast_convert_type(w, jnp.int32)
        return jnp.concatenate([ei, wb[None, :]], axis=0)

    epk_e = pack(edge_index, edge_w)
    epk_a1 = pack(aug1_edge_index, aug1_w)
    epk_a2 = pack(aug2_edge_index, aug2_w)

    b0r = b0.reshape(1, D)
    b1r = b1.reshape(1, D)
    a0r = jnp.broadcast_to(prelu0.reshape(1, 1), (1, D))
    a1r = jnp.broadcast_to(prelu1.reshape(1, 1), (1, D))
    g0r = gamma0.reshape(1, D)
    be0r = beta0.reshape(1, D)
    g1r = gamma1.reshape(1, D)
    be1r = beta1.reshape(1, D)
    lblT = lbl.reshape(2, N).T
    scal = jnp.stack([dgi_b[0], cl_b[0]])

    t1, t2 = pl.pallas_call(
        _mm2_body,
        grid=(NBLK,),
        in_specs=[_row_spec(), _row_spec(), _full_spec((D, D))],
        out_specs=[_row_spec(), _row_spec()],
        out_shape=[jax.ShapeDtypeStruct((N, D), _f32)] * 2,
    )(x1, x2, W0)

    p_s1e = _spmm(t1, epk_e)
    p_a1 = _spmm(t1, epk_a1)
    p_a2 = _spmm(t1, epk_a2)
    p_s2e = _spmm(t2, epk_e)

    v1, va, vb, v2, stats0 = pl.pallas_call(
        _kb_body,
        grid=(NBLK,),
        in_specs=[_pair_spec()] * 4 + [_full_spec((1, D)), _full_spec((1, D)),
                                       _full_spec((D, D))],
        out_specs=[_row_spec()] * 4 + [_full_spec((2, D))],
        out_shape=[jax.ShapeDtypeStruct((N, D), _f32)] * 4
        + [jax.ShapeDtypeStruct((2, D), _f32)],
    )(p_s1e, p_a1, p_a2, p_s2e, b0r, a0r, W1)

    vlp = pl.pallas_call(
        _blp_body,
        grid=(NBLK,),
        in_specs=[_pair_spec(), _full_spec((2, D))] + [_full_spec((1, D))] * 4
        + [_full_spec((D, D))],
        out_specs=_row_spec(),
        out_shape=jax.ShapeDtypeStruct((N, D), _f32),
    )(p_s1e, stats0, b0r, a0r, g0r, be0r, W1)

    q_s1e = _spmm(v1, epk_e)
    q_a1 = _spmm(va, epk_a1)
    q_a2 = _spmm(vb, epk_a2)
    q_s2e = _spmm(v2, epk_e)
    q_lp = _spmm(vlp, epk_e)

    stats1 = pl.pallas_call(
        _cs_body,
        grid=(NBLK,),
        in_specs=[_pair_spec()] * 4 + [_full_spec((1, D))] * 2,
        out_specs=_full_spec((8, D)),
        out_shape=jax.ShapeDtypeStruct((8, D), _f32),
    )(q_s1e, q_a1, q_a2, q_lp, b1r, a1r)

    logits3, lacc = pl.pallas_call(
        _cm_body,
        grid=(NBLK,),
        in_specs=[_pair_spec()] * 3 + [_full_spec((8, D))]
        + [_full_spec((1, D))] * 4
        + [_full_spec((1, D)), _full_spec((D, D)), _full_spec((1, D)),
           _full_spec((D, D)), _full_spec((1, D))]
        + [pl.BlockSpec((R, 2), lambda i: (i, 0)),
           pl.BlockSpec(memory_space=pltpu.SMEM)],
        out_specs=[_row_spec(), pl.BlockSpec(memory_space=pltpu.SMEM)],
        out_shape=[jax.ShapeDtypeStruct((N, D), _f32),
                   jax.ShapeDtypeStruct((2,), _f32)],
    )(q_s1e, q_s2e, q_lp, stats1, b1r, a1r, g1r, be1r,
      dgi_prompt, dgi_W, cl_prompt, cl_W, lp_prompt, lblT, scal)

    samp = jnp.concatenate(
        [sample.reshape(-1), jnp.zeros((24,), jnp.int32)])
    g1024 = _gather1024(logits3, samp)
    g3 = g1024[:1000].reshape(100, 10, D)
    f100 = logits3[:100]

    out = pl.pallas_call(
        _kd_body,
        in_specs=[pl.BlockSpec(memory_space=pltpu.VMEM),
                  pl.BlockSpec(memory_space=pltpu.VMEM),
                  pl.BlockSpec(memory_space=pltpu.SMEM)],
        out_specs=pl.BlockSpec(memory_space=pltpu.SMEM),
        out_shape=jax.ShapeDtypeStruct((1, 1), _f32),
    )(f100, g3, lacc)
    return out.reshape(())
